# baseline (speedup 1.0000x reference)
import functools

import jax
import jax.numpy as jnp
from jax import lax
from jax.experimental import pallas as pl
from jax.experimental.pallas import tpu as pltpu
from jax.experimental.pallas import tpu_sc as plsc

_F32 = jnp.float32
_BR = 256


def _sc_gather_rows(table, idx):
    V, D = table.shape
    B = idx.shape[0]
    info = plsc.get_sparse_core_info()
    NC, NS = info.num_cores, info.num_subcores
    NW = NC * NS
    b_per_w = B // NW
    mesh = plsc.VectorSubcoreMesh(core_axis_name="c", subcore_axis_name="s")

    @functools.partial(
        pl.kernel, mesh=mesh,
        out_type=jax.ShapeDtypeStruct((B, D), table.dtype),
        scratch_types=[
            pltpu.VMEM((b_per_w,), jnp.int32),
            pltpu.VMEM((b_per_w, D), table.dtype),
            pltpu.SemaphoreType.DMA,
        ],
    )
    def k(table_hbm, idx_hbm, out_hbm, idx_v, rows_v, sem):
        wid = lax.axis_index("s") * NC + lax.axis_index("c")
        base = wid * b_per_w
        pltpu.sync_copy(idx_hbm.at[pl.ds(base, b_per_w)], idx_v)
        pltpu.async_copy(table_hbm.at[idx_v], rows_v, sem).wait()
        pltpu.sync_copy(rows_v, out_hbm.at[pl.ds(base, b_per_w)])

    return k(table, idx)


def _proj(x, ws, a1, a2, block_rows=_BR):
    N, di = x.shape
    K = len(ws)

    def body(*refs):
        it = iter(refs)
        x_ref = next(it)
        w_refs = [next(it) for _ in range(K)]
        a1_ref, a2_ref = next(it), next(it)
        o_refs = [next(it) for _ in range(K)]
        s1_ref, s2_ref = next(it), next(it)
        xb = x_ref[...]
        for k, (w_ref, o_ref) in enumerate(zip(w_refs, o_refs)):
            hf = jnp.dot(xb, w_ref[...], preferred_element_type=_F32)
            o_ref[...] = hf
            if k == 0:
                s1_ref[...] = lax.dot_general(
                    hf, a1_ref[...], (((1,), (1,)), ((), ())),
                    preferred_element_type=_F32)
                s2_ref[...] = lax.dot_general(
                    a2_ref[...], hf, (((1,), (1,)), ((), ())),
                    preferred_element_type=_F32)

    in_specs = [pl.BlockSpec((block_rows, di), lambda i: (i, 0))]
    in_specs += [pl.BlockSpec(w.shape, lambda i: (0, 0)) for w in ws]
    in_specs += [pl.BlockSpec(a1.shape, lambda i: (0, 0)),
                 pl.BlockSpec(a2.shape, lambda i: (0, 0))]
    out_specs = [pl.BlockSpec((block_rows, w.shape[1]), lambda i: (i, 0))
                 for w in ws]
    out_specs += [pl.BlockSpec((block_rows, 1), lambda i: (i, 0)),
                  pl.BlockSpec((1, block_rows), lambda i: (0, i))]
    out_shape = [jax.ShapeDtypeStruct((N, w.shape[1]), _F32) for w in ws]
    out_shape += [jax.ShapeDtypeStruct((N, 1), _F32),
                  jax.ShapeDtypeStruct((1, N), _F32)]
    return list(pl.pallas_call(
        body,
        grid=(N // block_rows,),
        in_specs=in_specs,
        out_specs=out_specs,
        out_shape=out_shape,
    )(x, *ws, a1, a2))


def _attn_layer(phases, ha, h_spec, hda, hd_spec, hua, hu_spec,
                s1a, s2a, wnext, a1n, a2n, block_rows=_BR):
    do = ha.shape[1]
    K = len(wnext)
    nsteps = phases[-1]["hi"]
    rows_out = nsteps * block_rows

    def body(*refs):
        it = iter(refs)
        s1_ref, s2_ref = next(it), next(it)
        h_ref = next(it)
        hd_ref = next(it) if hda is not None else None
        hu_ref = next(it) if hua is not None else None
        L_refs = [next(it) for _ in phases]
        bd_refs = [next(it) if p["bd"] is not None else None for p in phases]
        bu_refs = [next(it) if p["bu"] is not None else None for p in phases]
        w_refs = [next(it) for _ in range(K)]
        a1_ref, a2_ref = next(it), next(it)
        o_refs = [next(it) for _ in range(K)]
        s1o_ref, s2o_ref = next(it), next(it)
        r_s = next(it)

        i = pl.program_id(0)
        s1 = s1_ref[...]
        for p_idx, p in enumerate(phases):
            N = p["L"].shape[0]

            @pl.when(jnp.logical_and(i >= p["lo"], i < p["hi"]))
            def _(p=p, p_idx=p_idx, N=N):
                e = s1 + s2_ref[:, p["s2_lo"]:p["s2_lo"] + N]
                e = jnp.where(e >= 0, e, 0.2 * e)
                e = jnp.where(L_refs[p_idx][...] != 0, e, -1e9)
                m = jnp.max(e, axis=1, keepdims=True)
                w = jnp.exp(e - m)
                den = jnp.sum(w, axis=1, keepdims=True)
                h = h_ref[p["h_lo"]:p["h_lo"] + N, :]
                acc = jnp.dot(w, h, preferred_element_type=_F32) / den
                if p["bd"] is not None:
                    npv = p["bd"].shape[0]
                    pd = hd_ref[p["pd_lo"]:p["pd_lo"] + npv, :]
                    acc += lax.dot_general(bd_refs[p_idx][...], pd,
                                           (((0,), (0,)), ((), ())),
                                           preferred_element_type=_F32)
                if p["bu"] is not None:
                    nnv = p["bu"].shape[1]
                    pu = hu_ref[p["pu_lo"]:p["pu_lo"] + nnv, :]
                    acc += jnp.dot(bu_refs[p_idx][...], pu,
                                   preferred_element_type=_F32)
                r_s[...] = jnp.maximum(acc, 0.0)

        r = r_s[...]
        for k, (w_ref, o_ref) in enumerate(zip(w_refs, o_refs)):
            hf = jnp.dot(r, w_ref[...], preferred_element_type=_F32)
            o_ref[...] = hf
            if k == 0:
                s1o_ref[...] = lax.dot_general(
                    hf, a1_ref[...], (((1,), (1,)), ((), ())),
                    preferred_element_type=_F32)
                s2o_ref[...] = lax.dot_general(
                    a2_ref[...], hf, (((1,), (1,)), ((), ())),
                    preferred_element_type=_F32)

    def s1_map(i):
        b = 0
        for p in phases:
            b = jnp.where(jnp.logical_and(i >= p["lo"], i < p["hi"]),
                          p["h_lo"] // block_rows + i - p["lo"], b)
        return (b, 0)

    in_specs = [
        pl.BlockSpec((block_rows, 1), s1_map),
        pl.BlockSpec((1, s2a.shape[1]), lambda i: (0, 0)),
        pl.BlockSpec((h_spec[0], do), lambda i, b=h_spec[1]: (b, 0)),
    ]
    args = [s1a, s2a, ha]
    if hda is not None:
        in_specs.append(
            pl.BlockSpec((hd_spec[0], do), lambda i, b=hd_spec[1]: (b, 0)))
        args.append(hda)
    if hua is not None:
        in_specs.append(
            pl.BlockSpec((hu_spec[0], do), lambda i, b=hu_spec[1]: (b, 0)))
        args.append(hua)
    for p in phases:
        N = p["L"].shape[0]
        lo, top = p["lo"], p["hi"] - p["lo"] - 1
        in_specs.append(pl.BlockSpec(
            (block_rows, N),
            lambda i, lo=lo, top=top: (jnp.clip(i - lo, 0, top), 0)))
        args.append(p["L"])
    for p in phases:
        if p["bd"] is not None:
            npv = p["bd"].shape[0]
            lo, top = p["lo"], p["hi"] - p["lo"] - 1
            in_specs.append(pl.BlockSpec(
                (npv, block_rows),
                lambda i, lo=lo, top=top: (0, jnp.clip(i - lo, 0, top))))
            args.append(p["bd"])
    for p in phases:
        if p["bu"] is not None:
            nnv = p["bu"].shape[1]
            lo, top = p["lo"], p["hi"] - p["lo"] - 1
            in_specs.append(pl.BlockSpec(
                (block_rows, nnv),
                lambda i, lo=lo, top=top: (jnp.clip(i - lo, 0, top), 0)))
            args.append(p["bu"])
    in_specs += [pl.BlockSpec(wk.shape, lambda i: (0, 0)) for wk in wnext]
    args += list(wnext)
    in_specs += [pl.BlockSpec(a1n.shape, lambda i: (0, 0)),
                 pl.BlockSpec(a2n.shape, lambda i: (0, 0))]
    args += [a1n, a2n]
    out_specs = [pl.BlockSpec((block_rows, wk.shape[1]), lambda i: (i, 0))
                 for wk in wnext]
    out_shape = [jax.ShapeDtypeStruct((rows_out, wk.shape[1]), _F32)
                 for wk in wnext]
    out_specs += [pl.BlockSpec((block_rows, 1), lambda i: (i, 0)),
                  pl.BlockSpec((1, block_rows), lambda i: (0, i))]
    out_shape += [jax.ShapeDtypeStruct((rows_out, 1), _F32),
                  jax.ShapeDtypeStruct((1, rows_out), _F32)]

    return list(pl.pallas_call(
        body,
        grid=(nsteps,),
        in_specs=in_specs,
        out_specs=out_specs,
        out_shape=out_shape,
        scratch_shapes=[pltpu.VMEM((block_rows, do), _F32)],
    )(*args))


def _final(Lg, idxc, s1a, s1_bidx, s2a, s2_bidx, h4a, h_bidx, Bg,
           hu4a, pu_bidx, wrel, brel, N, NP):
    B = Lg.shape[0]
    do = h4a.shape[1]
    C = wrel.shape[1]

    def body(Lg_ref, idx_ref, s1_ref, s2_ref, h0_ref, Bg_ref, pu_ref,
             wrel_ref, brel_ref, o_ref):
        cols = lax.broadcasted_iota(jnp.int32, (B, N), 1)
        oh = (cols == idx_ref[...]).astype(_F32)
        s1g = jnp.dot(oh, s1_ref[...], preferred_element_type=_F32)
        e = s1g + s2_ref[...]
        e = jnp.where(e >= 0, e, 0.2 * e)
        e = jnp.where(Lg_ref[...] != 0, e, -1e9)
        m = jnp.max(e, axis=1, keepdims=True)
        w = jnp.exp(e - m)
        den = jnp.sum(w, axis=1, keepdims=True)
        acc = jnp.dot(w, h0_ref[...], preferred_element_type=_F32) / den
        acc += jnp.dot(Bg_ref[...], pu_ref[...], preferred_element_type=_F32)
        acc = jnp.maximum(acc, 0.0)
        o_ref[...] = (jnp.dot(acc, wrel_ref[...], preferred_element_type=_F32)
                      + brel_ref[...])

    in_specs = [
        pl.BlockSpec(Lg.shape, lambda i: (0, 0)),
        pl.BlockSpec(idxc.shape, lambda i: (0, 0)),
        pl.BlockSpec((N, 1), lambda i: (s1_bidx, 0)),
        pl.BlockSpec((1, N), lambda i: (0, s2_bidx)),
        pl.BlockSpec((N, do), lambda i: (h_bidx, 0)),
        pl.BlockSpec(Bg.shape, lambda i: (0, 0)),
        pl.BlockSpec((NP, do), lambda i: (pu_bidx, 0)),
        pl.BlockSpec(wrel.shape, lambda i: (0, 0)),
        pl.BlockSpec(brel.shape, lambda i: (0, 0)),
    ]
    return pl.pallas_call(
        body,
        grid=(1,),
        in_specs=in_specs,
        out_specs=pl.BlockSpec((B, C), lambda i: (0, 0)),
        out_shape=jax.ShapeDtypeStruct((B, C), _F32),
    )(Lg, idxc, s1a, s2a, h4a, Bg, hu4a, wrel, brel)


def _split_a(lp):
    a = lp["a"]
    do = a.shape[0] // 2
    return a[:do].reshape(1, do), a[do:].reshape(1, do)


def kernel(emb0, emb1, emb2, emb3, lap0, lap1, lap2, lap3,
           bnd1, bnd2, bnd3, order, idx, rel, params):
    del order
    idx = idx.astype(jnp.int32)

    Lg = _sc_gather_rows(lap0, idx)
    Bg = _sc_gather_rows(bnd1, idx)

    l1, l2, l3, l4 = (params["l%d" % i] for i in (1, 2, 3, 4))
    a1p = _split_a(l1)
    a2p = _split_a(l2)
    a3p = _split_a(l3)
    a4p = _split_a(l4)

    xcat = jnp.concatenate([emb0, emb3, emb1, emb2], axis=0)
    ha, hda, hua, s1a, s2a = _proj(
        xcat, [l1["W"], l1["Wd"], l1["Wu"]], a1p[0], a1p[1])

    phases1 = [
        dict(lo=0, hi=4, L=lap0, h_lo=0, s2_lo=0,
             bd=None, pd_lo=0, bu=bnd1, pu_lo=2048),
        dict(lo=4, hi=8, L=lap3, h_lo=1024, s2_lo=1024,
             bd=bnd3, pd_lo=4096, bu=None, pu_lo=0),
        dict(lo=8, hi=16, L=lap1, h_lo=2048, s2_lo=2048,
             bd=bnd1, pd_lo=0, bu=bnd2, pu_lo=4096),
        dict(lo=16, hi=24, L=lap2, h_lo=4096, s2_lo=4096,
             bd=bnd2, pd_lo=2048, bu=bnd3, pu_lo=1024),
    ]
    h2, hd2, hu2, s12, s22 = _attn_layer(
        phases1, ha, (6144, 0), hda, (6144, 0), hua, (6144, 0),
        s1a, s2a, [l2["W"], l2["Wd"], l2["Wu"]], a2p[0], a2p[1])

    phases2 = [
        dict(lo=0, hi=8, L=lap1, h_lo=2048, s2_lo=2048,
             bd=bnd1, pd_lo=0, bu=bnd2, pu_lo=4096),
        dict(lo=8, hi=16, L=lap2, h_lo=4096, s2_lo=4096,
             bd=bnd2, pd_lo=2048, bu=bnd3, pu_lo=1024),
        dict(lo=16, hi=20, L=lap0, h_lo=0, s2_lo=0,
             bd=None, pd_lo=0, bu=bnd1, pu_lo=2048),
    ]
    h3, hd3, hu3, s13, s23 = _attn_layer(
        phases2, h2, (6144, 0), hd2, (4096, 0), hu2, (6144, 0),
        s12, s22, [l3["W"], l3["Wd"], l3["Wu"]], a3p[0], a3p[1])

    phases3 = [
        dict(lo=0, hi=8, L=lap1, h_lo=0, s2_lo=0,
             bd=bnd1, pd_lo=0, bu=bnd2, pu_lo=2048),
        dict(lo=8, hi=12, L=lap0, h_lo=4096, s2_lo=4096,
             bd=None, pd_lo=0, bu=bnd1, pu_lo=0),
    ]
    h4a, hu4a, s14, s24 = _attn_layer(
        phases3, h3, (5120, 0), hd3, (1024, 4), hu3, (4096, 0),
        s13, s23, [l4["W"], l4["Wu"]], a4p[0], a4p[1])

    rows = _final(Lg, idx.reshape(-1, 1), s14, 2, s24, 2, h4a, 2, Bg,
                  hu4a, 0, params["W_rel"], params["b_rel"].reshape(1, -1),
                  N=1024, NP=2048)

    nz = jnp.stack(jnp.nonzero(rel, size=rel.shape[0]), axis=1)
    return rows[nz]


# speedup vs baseline: 1.0798x; 1.0798x over previous
import functools

import jax
import jax.numpy as jnp
from jax import lax
from jax.experimental import pallas as pl
from jax.experimental.pallas import tpu as pltpu
from jax.experimental.pallas import tpu_sc as plsc

_F32 = jnp.float32
_BR = 512


def _sc_gather_rows(table, idx):
    V, D = table.shape
    B = idx.shape[0]
    info = plsc.get_sparse_core_info()
    NC, NS = info.num_cores, info.num_subcores
    NW = NC * NS
    b_per_w = B // NW
    mesh = plsc.VectorSubcoreMesh(core_axis_name="c", subcore_axis_name="s")

    @functools.partial(
        pl.kernel, mesh=mesh,
        out_type=jax.ShapeDtypeStruct((B, D), table.dtype),
        scratch_types=[
            pltpu.VMEM((b_per_w,), jnp.int32),
            pltpu.VMEM((b_per_w, D), table.dtype),
            pltpu.SemaphoreType.DMA,
        ],
    )
    def k(table_hbm, idx_hbm, out_hbm, idx_v, rows_v, sem):
        wid = lax.axis_index("s") * NC + lax.axis_index("c")
        base = wid * b_per_w
        pltpu.sync_copy(idx_hbm.at[pl.ds(base, b_per_w)], idx_v)
        pltpu.async_copy(table_hbm.at[idx_v], rows_v, sem).wait()
        pltpu.sync_copy(rows_v, out_hbm.at[pl.ds(base, b_per_w)])

    return k(table, idx)


def _proj(x, ws, a1, a2, block_rows=_BR):
    N, di = x.shape
    K = len(ws)

    def body(*refs):
        it = iter(refs)
        x_ref = next(it)
        w_refs = [next(it) for _ in range(K)]
        a1_ref, a2_ref = next(it), next(it)
        o_refs = [next(it) for _ in range(K)]
        s1_ref, s2_ref = next(it), next(it)
        xb = x_ref[...]
        for k, (w_ref, o_ref) in enumerate(zip(w_refs, o_refs)):
            hf = jnp.dot(xb, w_ref[...], preferred_element_type=_F32)
            o_ref[...] = hf
            if k == 0:
                s1_ref[...] = lax.dot_general(
                    hf, a1_ref[...], (((1,), (1,)), ((), ())),
                    preferred_element_type=_F32)
                s2_ref[...] = lax.dot_general(
                    a2_ref[...], hf, (((1,), (1,)), ((), ())),
                    preferred_element_type=_F32)

    in_specs = [pl.BlockSpec((block_rows, di), lambda i: (i, 0))]
    in_specs += [pl.BlockSpec(w.shape, lambda i: (0, 0)) for w in ws]
    in_specs += [pl.BlockSpec(a1.shape, lambda i: (0, 0)),
                 pl.BlockSpec(a2.shape, lambda i: (0, 0))]
    out_specs = [pl.BlockSpec((block_rows, w.shape[1]), lambda i: (i, 0))
                 for w in ws]
    out_specs += [pl.BlockSpec((block_rows, 1), lambda i: (i, 0)),
                  pl.BlockSpec((1, block_rows), lambda i: (0, i))]
    out_shape = [jax.ShapeDtypeStruct((N, w.shape[1]), _F32) for w in ws]
    out_shape += [jax.ShapeDtypeStruct((N, 1), _F32),
                  jax.ShapeDtypeStruct((1, N), _F32)]
    return list(pl.pallas_call(
        body,
        grid=(N // block_rows,),
        in_specs=in_specs,
        out_specs=out_specs,
        out_shape=out_shape,
    )(x, *ws, a1, a2))


def _attn(L, ha, s1a, s2a, off, bd, pda, doff, bu, pua, uoff,
          wnext, a1n=None, a2n=None, block_rows=_BR):
    N = L.shape[0]
    do = ha.shape[1]
    K = len(wnext)
    has_d = bd is not None
    has_u = bu is not None
    with_scores = a1n is not None

    def body(*refs):
        it = iter(refs)
        L_ref, h_ref, s1_ref, s2_ref = next(it), next(it), next(it), next(it)
        bd_ref = next(it) if has_d else None
        pd_ref = next(it) if has_d else None
        bu_ref = next(it) if has_u else None
        pu_ref = next(it) if has_u else None
        w_refs = [next(it) for _ in range(K)]
        if with_scores:
            a1_ref, a2_ref = next(it), next(it)
        o_refs = [next(it) for _ in range(K)]
        if with_scores:
            s1o_ref, s2o_ref = next(it), next(it)

        e = s1_ref[...] + s2_ref[...]
        e = jnp.where(e >= 0, e, 0.2 * e)
        e = jnp.where(L_ref[...] != 0, e, -1e9)
        m = jnp.max(e, axis=1, keepdims=True)
        w = jnp.exp(e - m)
        den = jnp.sum(w, axis=1, keepdims=True)
        acc = jnp.dot(w, h_ref[...], preferred_element_type=_F32) / den
        if has_d:
            acc += lax.dot_general(bd_ref[...], pd_ref[...],
                                   (((0,), (0,)), ((), ())),
                                   preferred_element_type=_F32)
        if has_u:
            acc += jnp.dot(bu_ref[...], pu_ref[...],
                           preferred_element_type=_F32)
        r = jnp.maximum(acc, 0.0)
        for k, (w_ref, o_ref) in enumerate(zip(w_refs, o_refs)):
            hf = jnp.dot(r, w_ref[...], preferred_element_type=_F32)
            o_ref[...] = hf
            if with_scores and k == 0:
                s1o_ref[...] = lax.dot_general(
                    hf, a1_ref[...], (((1,), (1,)), ((), ())),
                    preferred_element_type=_F32)
                s2o_ref[...] = lax.dot_general(
                    a2_ref[...], hf, (((1,), (1,)), ((), ())),
                    preferred_element_type=_F32)

    hb = off // N
    sb = off // block_rows
    in_specs = [
        pl.BlockSpec((block_rows, N), lambda i: (i, 0)),
        pl.BlockSpec((N, do), lambda i, b=hb: (b, 0)),
        pl.BlockSpec((block_rows, 1), lambda i, b=sb: (b + i, 0)),
        pl.BlockSpec((1, N), lambda i, b=hb: (0, b)),
    ]
    args = [L, ha, s1a, s2a]
    if has_d:
        npv = bd.shape[0]
        db = doff // npv
        in_specs += [pl.BlockSpec((npv, block_rows), lambda i: (0, i)),
                     pl.BlockSpec((npv, do), lambda i, b=db: (b, 0))]
        args += [bd, pda]
    if has_u:
        nnv = bu.shape[1]
        ub = uoff // nnv
        in_specs += [pl.BlockSpec((block_rows, nnv), lambda i: (i, 0)),
                     pl.BlockSpec((nnv, do), lambda i, b=ub: (b, 0))]
        args += [bu, pua]
    in_specs += [pl.BlockSpec(wk.shape, lambda i: (0, 0)) for wk in wnext]
    args += list(wnext)
    out_specs = [pl.BlockSpec((block_rows, wk.shape[1]), lambda i: (i, 0))
                 for wk in wnext]
    out_shape = [jax.ShapeDtypeStruct((N, wk.shape[1]), _F32)
                 for wk in wnext]
    if with_scores:
        in_specs += [pl.BlockSpec(a1n.shape, lambda i: (0, 0)),
                     pl.BlockSpec(a2n.shape, lambda i: (0, 0))]
        args += [a1n, a2n]
        out_specs += [pl.BlockSpec((block_rows, 1), lambda i: (i, 0)),
                      pl.BlockSpec((1, block_rows), lambda i: (0, i))]
        out_shape += [jax.ShapeDtypeStruct((N, 1), _F32),
                      jax.ShapeDtypeStruct((1, N), _F32)]

    return list(pl.pallas_call(
        body,
        grid=(N // block_rows,),
        in_specs=in_specs,
        out_specs=out_specs,
        out_shape=out_shape,
    )(*args))


def _final(Lg, idxc, s1, s2, h0, Bg, pu, wrel, brel):
    B = Lg.shape[0]
    N, do = h0.shape
    C = wrel.shape[1]

    def body(Lg_ref, idx_ref, s1_ref, s2_ref, h0_ref, Bg_ref, pu_ref,
             wrel_ref, brel_ref, o_ref):
        cols = lax.broadcasted_iota(jnp.int32, (B, N), 1)
        oh = (cols == idx_ref[...]).astype(_F32)
        s1g = jnp.dot(oh, s1_ref[...], preferred_element_type=_F32)
        e = s1g + s2_ref[...]
        e = jnp.where(e >= 0, e, 0.2 * e)
        e = jnp.where(Lg_ref[...] != 0, e, -1e9)
        m = jnp.max(e, axis=1, keepdims=True)
        w = jnp.exp(e - m)
        den = jnp.sum(w, axis=1, keepdims=True)
        acc = jnp.dot(w, h0_ref[...], preferred_element_type=_F32) / den
        acc += jnp.dot(Bg_ref[...], pu_ref[...], preferred_element_type=_F32)
        acc = jnp.maximum(acc, 0.0)
        o_ref[...] = (jnp.dot(acc, wrel_ref[...], preferred_element_type=_F32)
                      + brel_ref[...])

    return pl.pallas_call(
        body,
        out_shape=jax.ShapeDtypeStruct((B, C), _F32),
    )(Lg, idxc, s1, s2, h0, Bg, pu, wrel, brel)


def _split_a(lp):
    a = lp["a"]
    do = a.shape[0] // 2
    return a[:do].reshape(1, do), a[do:].reshape(1, do)


def kernel(emb0, emb1, emb2, emb3, lap0, lap1, lap2, lap3,
           bnd1, bnd2, bnd3, order, idx, rel, params):
    del order
    idx = idx.astype(jnp.int32)

    Lg = _sc_gather_rows(lap0, idx)
    Bg = _sc_gather_rows(bnd1, idx)

    laps = [lap0, lap1, lap2, lap3]
    bnds = [None, bnd1, bnd2, bnd3]
    l1, l2, l3, l4 = (params["l%d" % i] for i in (1, 2, 3, 4))
    a2p = _split_a(l2)
    a3p = _split_a(l3)
    a4p = _split_a(l4)
    wmap = {"h": "W", "d": "Wd", "u": "Wu"}

    so = (0, 3, 1, 2)
    embs = [emb0, emb1, emb2, emb3]
    offs = {}
    off = 0
    for i in so:
        offs[i] = off
        off += embs[i].shape[0]
    xcat = jnp.concatenate([embs[i] for i in so], axis=0)
    a1, a2 = _split_a(l1)
    ha, hda, hua, s1a, s2a = _proj(xcat, [l1["W"], l1["Wd"], l1["Wu"]], a1, a2)

    kinds = {0: "h d", 1: "h d u", 2: "h u", 3: "u"}
    h2, hd2, hu2, s12, s22 = {}, {}, {}, {}, {}
    for i in (0, 1, 2, 3):
        toks = kinds[i].split()
        ws = [l2[wmap[t]] for t in toks]
        sc = "h" in toks
        outs = _attn(laps[i], ha, s1a, s2a, offs[i],
                     bnds[i] if i > 0 else None, hda,
                     offs[i - 1] if i > 0 else 0,
                     bnds[i + 1] if i < 3 else None, hua,
                     offs[i + 1] if i < 3 else 0,
                     ws, a2p[0] if sc else None, a2p[1] if sc else None)
        dsts = {"h": h2, "d": hd2, "u": hu2}
        for t, o in zip(toks, outs[:len(toks)]):
            dsts[t][i] = o
        if sc:
            s12[i], s22[i] = outs[-2], outs[-1]

    kinds = {0: "h d", 1: "h u", 2: "u"}
    h3, hd3, hu3, s13, s23 = {}, {}, {}, {}, {}
    for i in (0, 1, 2):
        toks = kinds[i].split()
        ws = [l3[wmap[t]] for t in toks]
        sc = "h" in toks
        outs = _attn(laps[i], h2[i], s12[i], s22[i], 0,
                     bnds[i] if i > 0 else None, hd2.get(i - 1), 0,
                     bnds[i + 1], hu2.get(i + 1), 0,
                     ws, a3p[0] if sc else None, a3p[1] if sc else None)
        dsts = {"h": h3, "d": hd3, "u": hu3}
        for t, o in zip(toks, outs[:len(toks)]):
            dsts[t][i] = o
        if sc:
            s13[i], s23[i] = outs[-2], outs[-1]

    h40, s14, s24 = _attn(laps[0], h3[0], s13[0], s23[0], 0,
                          None, None, 0, bnds[1], hu3[1], 0,
                          [l4["W"]], a4p[0], a4p[1])
    (pu41,) = _attn(laps[1], h3[1], s13[1], s23[1], 0,
                    bnds[1], hd3[0], 0, bnds[2], hu3[2], 0,
                    [l4["Wu"]])

    rows = _final(Lg, idx.reshape(-1, 1), s14, s24, h40, Bg, pu41,
                  params["W_rel"], params["b_rel"].reshape(1, -1))

    nz = jnp.stack(jnp.nonzero(rel, size=rel.shape[0]), axis=1)
    return rows[nz]


# speedup vs baseline: 1.0965x; 1.0155x over previous
import functools

import jax
import jax.numpy as jnp
from jax import lax
from jax.experimental import pallas as pl
from jax.experimental.pallas import tpu as pltpu
from jax.experimental.pallas import tpu_sc as plsc

_F32 = jnp.float32
_BR = 512


def _sc_gather_rows(table, idx):
    V, D = table.shape
    B = idx.shape[0]
    info = plsc.get_sparse_core_info()
    NC, NS = info.num_cores, info.num_subcores
    NW = NC * NS
    b_per_w = B // NW
    mesh = plsc.VectorSubcoreMesh(core_axis_name="c", subcore_axis_name="s")

    @functools.partial(
        pl.kernel, mesh=mesh,
        out_type=jax.ShapeDtypeStruct((B, D), table.dtype),
        scratch_types=[
            pltpu.VMEM((b_per_w,), jnp.int32),
            pltpu.VMEM((b_per_w, D), table.dtype),
            pltpu.SemaphoreType.DMA,
        ],
    )
    def k(table_hbm, idx_hbm, out_hbm, idx_v, rows_v, sem):
        wid = lax.axis_index("s") * NC + lax.axis_index("c")
        base = wid * b_per_w
        pltpu.sync_copy(idx_hbm.at[pl.ds(base, b_per_w)], idx_v)
        pltpu.async_copy(table_hbm.at[idx_v], rows_v, sem).wait()
        pltpu.sync_copy(rows_v, out_hbm.at[pl.ds(base, b_per_w)])

    return k(table, idx)


def _proj(x, ws, a1, a2, block_rows=_BR):
    N, di = x.shape
    K = len(ws)

    def body(*refs):
        it = iter(refs)
        x_ref = next(it)
        w_refs = [next(it) for _ in range(K)]
        a1_ref, a2_ref = next(it), next(it)
        o_refs = [next(it) for _ in range(K)]
        s1_ref, s2_ref = next(it), next(it)
        xb = x_ref[...]
        for k, (w_ref, o_ref) in enumerate(zip(w_refs, o_refs)):
            hf = jnp.dot(xb, w_ref[...], preferred_element_type=_F32)
            o_ref[...] = hf
            if k == 0:
                s1_ref[...] = lax.dot_general(
                    hf, a1_ref[...], (((1,), (1,)), ((), ())),
                    preferred_element_type=_F32)
                s2_ref[...] = lax.dot_general(
                    a2_ref[...], hf, (((1,), (1,)), ((), ())),
                    preferred_element_type=_F32)

    in_specs = [pl.BlockSpec((block_rows, di), lambda i: (i, 0))]
    in_specs += [pl.BlockSpec(w.shape, lambda i: (0, 0)) for w in ws]
    in_specs += [pl.BlockSpec(a1.shape, lambda i: (0, 0)),
                 pl.BlockSpec(a2.shape, lambda i: (0, 0))]
    out_specs = [pl.BlockSpec((block_rows, w.shape[1]), lambda i: (i, 0))
                 for w in ws]
    out_specs += [pl.BlockSpec((block_rows, 1), lambda i: (i, 0)),
                  pl.BlockSpec((1, block_rows), lambda i: (0, i))]
    out_shape = [jax.ShapeDtypeStruct((N, w.shape[1]), _F32) for w in ws]
    out_shape += [jax.ShapeDtypeStruct((N, 1), _F32),
                  jax.ShapeDtypeStruct((1, N), _F32)]
    return list(pl.pallas_call(
        body,
        grid=(N // block_rows,),
        in_specs=in_specs,
        out_specs=out_specs,
        out_shape=out_shape,
    )(x, *ws, a1, a2))


def _attn(L, ha, s1a, s2a, off, bd, pda, doff, bu, pua, uoff,
          wnext, a1n=None, a2n=None, bg=None, emit_out=True, block_rows=_BR):
    N = L.shape[0]
    do = ha.shape[1]
    K = len(wnext)
    has_d = bd is not None
    has_u = bu is not None
    with_scores = a1n is not None
    with_bg = bg is not None

    def body(*refs):
        it = iter(refs)
        L_ref, h_ref, s1_ref, s2_ref = next(it), next(it), next(it), next(it)
        bd_ref = next(it) if has_d else None
        pd_ref = next(it) if has_d else None
        bu_ref = next(it) if has_u else None
        pu_ref = next(it) if has_u else None
        w_refs = [next(it) for _ in range(K)]
        if with_scores:
            a1_ref, a2_ref = next(it), next(it)
        bg_ref = next(it) if with_bg else None
        o_refs = [next(it) for _ in range(K)] if emit_out else []
        if with_scores:
            s1o_ref, s2o_ref = next(it), next(it)
        bgo_ref = next(it) if with_bg else None

        e = s1_ref[...] + s2_ref[...]
        e = jnp.where(e >= 0, e, 0.2 * e)
        e = jnp.where(L_ref[...] != 0, e, -1e9)
        m = jnp.max(e, axis=1, keepdims=True)
        w = jnp.exp(e - m)
        den = jnp.sum(w, axis=1, keepdims=True)
        acc = jnp.dot(w, h_ref[...], preferred_element_type=_F32) / den
        if has_d:
            acc += lax.dot_general(bd_ref[...], pd_ref[...],
                                   (((0,), (0,)), ((), ())),
                                   preferred_element_type=_F32)
        if has_u:
            acc += jnp.dot(bu_ref[...], pu_ref[...],
                           preferred_element_type=_F32)
        r = jnp.maximum(acc, 0.0)
        for k, w_ref in enumerate(w_refs):
            hf = jnp.dot(r, w_ref[...], preferred_element_type=_F32)
            if emit_out:
                o_refs[k][...] = hf
            if with_scores and k == 0:
                s1o_ref[...] = lax.dot_general(
                    hf, a1_ref[...], (((1,), (1,)), ((), ())),
                    preferred_element_type=_F32)
                s2o_ref[...] = lax.dot_general(
                    a2_ref[...], hf, (((1,), (1,)), ((), ())),
                    preferred_element_type=_F32)
            if with_bg and k == 0:
                part = jnp.dot(bg_ref[...], hf, preferred_element_type=_F32)
                i = pl.program_id(0)
                bgo_ref[...] = jnp.where(i == 0, part, bgo_ref[...] + part)

    hb = off // N
    sb = off // block_rows
    in_specs = [
        pl.BlockSpec((block_rows, N), lambda i: (i, 0)),
        pl.BlockSpec((N, do), lambda i, b=hb: (b, 0)),
        pl.BlockSpec((block_rows, 1), lambda i, b=sb: (b + i, 0)),
        pl.BlockSpec((1, N), lambda i, b=hb: (0, b)),
    ]
    args = [L, ha, s1a, s2a]
    if has_d:
        npv = bd.shape[0]
        db = doff // npv
        in_specs += [pl.BlockSpec((npv, block_rows), lambda i: (0, i)),
                     pl.BlockSpec((npv, do), lambda i, b=db: (b, 0))]
        args += [bd, pda]
    if has_u:
        nnv = bu.shape[1]
        ub = uoff // nnv
        in_specs += [pl.BlockSpec((block_rows, nnv), lambda i: (i, 0)),
                     pl.BlockSpec((nnv, do), lambda i, b=ub: (b, 0))]
        args += [bu, pua]
    in_specs += [pl.BlockSpec(wk.shape, lambda i: (0, 0)) for wk in wnext]
    args += list(wnext)
    if with_scores:
        in_specs += [pl.BlockSpec(a1n.shape, lambda i: (0, 0)),
                     pl.BlockSpec(a2n.shape, lambda i: (0, 0))]
        args += [a1n, a2n]
    if with_bg:
        in_specs += [pl.BlockSpec((bg.shape[0], block_rows),
                                  lambda i: (0, i))]
        args += [bg]
    out_specs, out_shape = [], []
    if emit_out:
        out_specs += [pl.BlockSpec((block_rows, wk.shape[1]),
                                   lambda i: (i, 0)) for wk in wnext]
        out_shape += [jax.ShapeDtypeStruct((N, wk.shape[1]), _F32)
                      for wk in wnext]
    if with_scores:
        out_specs += [pl.BlockSpec((block_rows, 1), lambda i: (i, 0)),
                      pl.BlockSpec((1, block_rows), lambda i: (0, i))]
        out_shape += [jax.ShapeDtypeStruct((N, 1), _F32),
                      jax.ShapeDtypeStruct((1, N), _F32)]
    if with_bg:
        out_specs += [pl.BlockSpec((bg.shape[0], wnext[0].shape[1]),
                                   lambda i: (0, 0))]
        out_shape += [jax.ShapeDtypeStruct((bg.shape[0], wnext[0].shape[1]),
                                           _F32)]

    return list(pl.pallas_call(
        body,
        grid=(N // block_rows,),
        in_specs=in_specs,
        out_specs=out_specs,
        out_shape=out_shape,
    )(*args))


def _final(Lg, idxc, s1, s2, h0, bgpu, wrel, brel):
    B = Lg.shape[0]
    N, do = h0.shape
    C = wrel.shape[1]

    def body(Lg_ref, idx_ref, s1_ref, s2_ref, h0_ref, bgpu_ref,
             wrel_ref, brel_ref, o_ref):
        cols = lax.broadcasted_iota(jnp.int32, (B, N), 1)
        oh = (cols == idx_ref[...]).astype(_F32)
        s1g = jnp.dot(oh, s1_ref[...], preferred_element_type=_F32)
        e = s1g + s2_ref[...]
        e = jnp.where(e >= 0, e, 0.2 * e)
        e = jnp.where(Lg_ref[...] != 0, e, -1e9)
        m = jnp.max(e, axis=1, keepdims=True)
        w = jnp.exp(e - m)
        den = jnp.sum(w, axis=1, keepdims=True)
        acc = jnp.dot(w, h0_ref[...], preferred_element_type=_F32) / den
        acc += bgpu_ref[...]
        acc = jnp.maximum(acc, 0.0)
        o_ref[...] = (jnp.dot(acc, wrel_ref[...], preferred_element_type=_F32)
                      + brel_ref[...])

    return pl.pallas_call(
        body,
        out_shape=jax.ShapeDtypeStruct((B, C), _F32),
    )(Lg, idxc, s1, s2, h0, bgpu, wrel, brel)


def _split_a(lp):
    a = lp["a"]
    do = a.shape[0] // 2
    return a[:do].reshape(1, do), a[do:].reshape(1, do)


def kernel(emb0, emb1, emb2, emb3, lap0, lap1, lap2, lap3,
           bnd1, bnd2, bnd3, order, idx, rel, params):
    del order
    idx = idx.astype(jnp.int32)

    Lg = _sc_gather_rows(lap0, idx)
    Bg = _sc_gather_rows(bnd1, idx)

    laps = [lap0, lap1, lap2, lap3]
    bnds = [None, bnd1, bnd2, bnd3]
    l1, l2, l3, l4 = (params["l%d" % i] for i in (1, 2, 3, 4))
    a2p = _split_a(l2)
    a3p = _split_a(l3)
    a4p = _split_a(l4)
    wmap = {"h": "W", "d": "Wd", "u": "Wu"}

    so = (0, 3, 1, 2)
    embs = [emb0, emb1, emb2, emb3]
    offs = {}
    off = 0
    for i in so:
        offs[i] = off
        off += embs[i].shape[0]
    xcat = jnp.concatenate([embs[i] for i in so], axis=0)
    a1, a2 = _split_a(l1)
    ha, hda, hua, s1a, s2a = _proj(xcat, [l1["W"], l1["Wd"], l1["Wu"]], a1, a2)

    kinds = {0: "h d", 1: "h d u", 2: "h u", 3: "u"}
    h2, hd2, hu2, s12, s22 = {}, {}, {}, {}, {}
    for i in (0, 1, 2, 3):
        toks = kinds[i].split()
        ws = [l2[wmap[t]] for t in toks]
        sc = "h" in toks
        outs = _attn(laps[i], ha, s1a, s2a, offs[i],
                     bnds[i] if i > 0 else None, hda,
                     offs[i - 1] if i > 0 else 0,
                     bnds[i + 1] if i < 3 else None, hua,
                     offs[i + 1] if i < 3 else 0,
                     ws, a2p[0] if sc else None, a2p[1] if sc else None)
        dsts = {"h": h2, "d": hd2, "u": hu2}
        for t, o in zip(toks, outs[:len(toks)]):
            dsts[t][i] = o
        if sc:
            s12[i], s22[i] = outs[-2], outs[-1]

    kinds = {0: "h d", 1: "h u", 2: "u"}
    h3, hd3, hu3, s13, s23 = {}, {}, {}, {}, {}
    for i in (0, 1, 2):
        toks = kinds[i].split()
        ws = [l3[wmap[t]] for t in toks]
        sc = "h" in toks
        outs = _attn(laps[i], h2[i], s12[i], s22[i], 0,
                     bnds[i] if i > 0 else None, hd2.get(i - 1), 0,
                     bnds[i + 1], hu2.get(i + 1), 0,
                     ws, a3p[0] if sc else None, a3p[1] if sc else None)
        dsts = {"h": h3, "d": hd3, "u": hu3}
        for t, o in zip(toks, outs[:len(toks)]):
            dsts[t][i] = o
        if sc:
            s13[i], s23[i] = outs[-2], outs[-1]

    h40, s14, s24 = _attn(laps[0], h3[0], s13[0], s23[0], 0,
                          None, None, 0, bnds[1], hu3[1], 0,
                          [l4["W"]], a4p[0], a4p[1])
    (bgpu,) = _attn(laps[1], h3[1], s13[1], s23[1], 0,
                    bnds[1], hd3[0], 0, bnds[2], hu3[2], 0,
                    [l4["Wu"]], bg=Bg, emit_out=False)

    rows = _final(Lg, idx.reshape(-1, 1), s14, s24, h40, bgpu,
                  params["W_rel"], params["b_rel"].reshape(1, -1))

    nz = jnp.stack(jnp.nonzero(rel, size=rel.shape[0]), axis=1)
    return rows[nz]


# speedup vs baseline: 1.0975x; 1.0009x over previous
import functools

import jax
import jax.numpy as jnp
from jax import lax
from jax.experimental import pallas as pl
from jax.experimental.pallas import tpu as pltpu
from jax.experimental.pallas import tpu_sc as plsc

_F32 = jnp.float32
_BR = 512


def _sc_gather_rows2(t1, t2, idx):
    B = idx.shape[0]
    D1, D2 = t1.shape[1], t2.shape[1]
    info = plsc.get_sparse_core_info()
    NC, NS = info.num_cores, info.num_subcores
    NW = NC * NS
    b_per_w = B // NW
    mesh = plsc.VectorSubcoreMesh(core_axis_name="c", subcore_axis_name="s")

    @functools.partial(
        pl.kernel, mesh=mesh,
        out_type=[jax.ShapeDtypeStruct((B, D1), t1.dtype),
                  jax.ShapeDtypeStruct((B, D2), t2.dtype)],
        scratch_types=[
            pltpu.VMEM((b_per_w,), jnp.int32),
            pltpu.VMEM((b_per_w, D1), t1.dtype),
            pltpu.VMEM((b_per_w, D2), t2.dtype),
            pltpu.SemaphoreType.DMA,
            pltpu.SemaphoreType.DMA,
        ],
    )
    def k(t1_hbm, t2_hbm, idx_hbm, o1_hbm, o2_hbm,
          idx_v, r1_v, r2_v, sem1, sem2):
        wid = lax.axis_index("s") * NC + lax.axis_index("c")
        base = wid * b_per_w
        pltpu.sync_copy(idx_hbm.at[pl.ds(base, b_per_w)], idx_v)
        c1 = pltpu.async_copy(t1_hbm.at[idx_v], r1_v, sem1)
        c2 = pltpu.async_copy(t2_hbm.at[idx_v], r2_v, sem2)
        c1.wait()
        c2.wait()
        pltpu.sync_copy(r1_v, o1_hbm.at[pl.ds(base, b_per_w)])
        pltpu.sync_copy(r2_v, o2_hbm.at[pl.ds(base, b_per_w)])

    return k(t1, t2, idx)


def _proj(x, ws, a1, a2, block_rows=_BR):
    N, di = x.shape
    K = len(ws)

    def body(*refs):
        it = iter(refs)
        x_ref = next(it)
        w_refs = [next(it) for _ in range(K)]
        a1_ref, a2_ref = next(it), next(it)
        o_refs = [next(it) for _ in range(K)]
        s1_ref, s2_ref = next(it), next(it)
        xb = x_ref[...]
        for k, (w_ref, o_ref) in enumerate(zip(w_refs, o_refs)):
            hf = jnp.dot(xb, w_ref[...], preferred_element_type=_F32)
            o_ref[...] = hf
            if k == 0:
                s1_ref[...] = lax.dot_general(
                    hf, a1_ref[...], (((1,), (1,)), ((), ())),
                    preferred_element_type=_F32)
                s2_ref[...] = lax.dot_general(
                    a2_ref[...], hf, (((1,), (1,)), ((), ())),
                    preferred_element_type=_F32)

    in_specs = [pl.BlockSpec((block_rows, di), lambda i: (i, 0))]
    in_specs += [pl.BlockSpec(w.shape, lambda i: (0, 0)) for w in ws]
    in_specs += [pl.BlockSpec(a1.shape, lambda i: (0, 0)),
                 pl.BlockSpec(a2.shape, lambda i: (0, 0))]
    out_specs = [pl.BlockSpec((block_rows, w.shape[1]), lambda i: (i, 0))
                 for w in ws]
    out_specs += [pl.BlockSpec((block_rows, 1), lambda i: (i, 0)),
                  pl.BlockSpec((1, block_rows), lambda i: (0, i))]
    out_shape = [jax.ShapeDtypeStruct((N, w.shape[1]), _F32) for w in ws]
    out_shape += [jax.ShapeDtypeStruct((N, 1), _F32),
                  jax.ShapeDtypeStruct((1, N), _F32)]
    return list(pl.pallas_call(
        body,
        grid=(N // block_rows,),
        in_specs=in_specs,
        out_specs=out_specs,
        out_shape=out_shape,
    )(x, *ws, a1, a2))


def _attn(L, ha, s1a, s2a, off, bd, pda, doff, bu, pua, uoff,
          wnext, a1n=None, a2n=None, bg=None, emit_out=True, block_rows=_BR):
    N = L.shape[0]
    do = ha.shape[1]
    K = len(wnext)
    has_d = bd is not None
    has_u = bu is not None
    with_scores = a1n is not None
    with_bg = bg is not None

    def body(*refs):
        it = iter(refs)
        L_ref, h_ref, s1_ref, s2_ref = next(it), next(it), next(it), next(it)
        bd_ref = next(it) if has_d else None
        pd_ref = next(it) if has_d else None
        bu_ref = next(it) if has_u else None
        pu_ref = next(it) if has_u else None
        w_refs = [next(it) for _ in range(K)]
        if with_scores:
            a1_ref, a2_ref = next(it), next(it)
        bg_ref = next(it) if with_bg else None
        o_refs = [next(it) for _ in range(K)] if emit_out else []
        if with_scores:
            s1o_ref, s2o_ref = next(it), next(it)
        bgo_ref = next(it) if with_bg else None

        e = s1_ref[...] + s2_ref[...]
        e = jnp.where(e >= 0, e, 0.2 * e)
        e = jnp.where(L_ref[...] != 0, e, -1e9)
        m = jnp.max(e, axis=1, keepdims=True)
        w = jnp.exp(e - m)
        den = jnp.sum(w, axis=1, keepdims=True)
        acc = jnp.dot(w, h_ref[...], preferred_element_type=_F32) / den
        if has_d:
            acc += lax.dot_general(bd_ref[...], pd_ref[...],
                                   (((0,), (0,)), ((), ())),
                                   preferred_element_type=_F32)
        if has_u:
            acc += jnp.dot(bu_ref[...], pu_ref[...],
                           preferred_element_type=_F32)
        r = jnp.maximum(acc, 0.0)
        for k, w_ref in enumerate(w_refs):
            hf = jnp.dot(r, w_ref[...], preferred_element_type=_F32)
            if emit_out:
                o_refs[k][...] = hf
            if with_scores and k == 0:
                s1o_ref[...] = lax.dot_general(
                    hf, a1_ref[...], (((1,), (1,)), ((), ())),
                    preferred_element_type=_F32)
                s2o_ref[...] = lax.dot_general(
                    a2_ref[...], hf, (((1,), (1,)), ((), ())),
                    preferred_element_type=_F32)
            if with_bg and k == 0:
                part = jnp.dot(bg_ref[...], hf, preferred_element_type=_F32)
                i = pl.program_id(0)
                bgo_ref[...] = jnp.where(i == 0, part, bgo_ref[...] + part)

    hb = off // N
    sb = off // block_rows
    in_specs = [
        pl.BlockSpec((block_rows, N), lambda i: (i, 0)),
        pl.BlockSpec((N, do), lambda i, b=hb: (b, 0)),
        pl.BlockSpec((block_rows, 1), lambda i, b=sb: (b + i, 0)),
        pl.BlockSpec((1, N), lambda i, b=hb: (0, b)),
    ]
    args = [L, ha, s1a, s2a]
    if has_d:
        npv = bd.shape[0]
        db = doff // npv
        in_specs += [pl.BlockSpec((npv, block_rows), lambda i: (0, i)),
                     pl.BlockSpec((npv, do), lambda i, b=db: (b, 0))]
        args += [bd, pda]
    if has_u:
        nnv = bu.shape[1]
        ub = uoff // nnv
        in_specs += [pl.BlockSpec((block_rows, nnv), lambda i: (i, 0)),
                     pl.BlockSpec((nnv, do), lambda i, b=ub: (b, 0))]
        args += [bu, pua]
    in_specs += [pl.BlockSpec(wk.shape, lambda i: (0, 0)) for wk in wnext]
    args += list(wnext)
    if with_scores:
        in_specs += [pl.BlockSpec(a1n.shape, lambda i: (0, 0)),
                     pl.BlockSpec(a2n.shape, lambda i: (0, 0))]
        args += [a1n, a2n]
    if with_bg:
        in_specs += [pl.BlockSpec((bg.shape[0], block_rows),
                                  lambda i: (0, i))]
        args += [bg]
    out_specs, out_shape = [], []
    if emit_out:
        out_specs += [pl.BlockSpec((block_rows, wk.shape[1]),
                                   lambda i: (i, 0)) for wk in wnext]
        out_shape += [jax.ShapeDtypeStruct((N, wk.shape[1]), _F32)
                      for wk in wnext]
    if with_scores:
        out_specs += [pl.BlockSpec((block_rows, 1), lambda i: (i, 0)),
                      pl.BlockSpec((1, block_rows), lambda i: (0, i))]
        out_shape += [jax.ShapeDtypeStruct((N, 1), _F32),
                      jax.ShapeDtypeStruct((1, N), _F32)]
    if with_bg:
        out_specs += [pl.BlockSpec((bg.shape[0], wnext[0].shape[1]),
                                   lambda i: (0, 0))]
        out_shape += [jax.ShapeDtypeStruct((bg.shape[0], wnext[0].shape[1]),
                                           _F32)]

    return list(pl.pallas_call(
        body,
        grid=(N // block_rows,),
        in_specs=in_specs,
        out_specs=out_specs,
        out_shape=out_shape,
    )(*args))


def _final(Lg, idxc, s1, s2, h0, bgpu, wrel, brel):
    B = Lg.shape[0]
    N, do = h0.shape
    C = wrel.shape[1]

    def body(Lg_ref, idx_ref, s1_ref, s2_ref, h0_ref, bgpu_ref,
             wrel_ref, brel_ref, o_ref):
        cols = lax.broadcasted_iota(jnp.int32, (B, N), 1)
        oh = (cols == idx_ref[...]).astype(_F32)
        s1g = jnp.dot(oh, s1_ref[...], preferred_element_type=_F32)
        e = s1g + s2_ref[...]
        e = jnp.where(e >= 0, e, 0.2 * e)
        e = jnp.where(Lg_ref[...] != 0, e, -1e9)
        m = jnp.max(e, axis=1, keepdims=True)
        w = jnp.exp(e - m)
        den = jnp.sum(w, axis=1, keepdims=True)
        acc = jnp.dot(w, h0_ref[...], preferred_element_type=_F32) / den
        acc += bgpu_ref[...]
        acc = jnp.maximum(acc, 0.0)
        o_ref[...] = (jnp.dot(acc, wrel_ref[...], preferred_element_type=_F32)
                      + brel_ref[...])

    return pl.pallas_call(
        body,
        out_shape=jax.ShapeDtypeStruct((B, C), _F32),
    )(Lg, idxc, s1, s2, h0, bgpu, wrel, brel)


def _split_a(lp):
    a = lp["a"]
    do = a.shape[0] // 2
    return a[:do].reshape(1, do), a[do:].reshape(1, do)


def kernel(emb0, emb1, emb2, emb3, lap0, lap1, lap2, lap3,
           bnd1, bnd2, bnd3, order, idx, rel, params):
    del order
    idx = idx.astype(jnp.int32)

    Lg, Bg = _sc_gather_rows2(lap0, bnd1, idx)

    laps = [lap0, lap1, lap2, lap3]
    bnds = [None, bnd1, bnd2, bnd3]
    l1, l2, l3, l4 = (params["l%d" % i] for i in (1, 2, 3, 4))
    a2p = _split_a(l2)
    a3p = _split_a(l3)
    a4p = _split_a(l4)
    wmap = {"h": "W", "d": "Wd", "u": "Wu"}

    so = (0, 3, 1, 2)
    embs = [emb0, emb1, emb2, emb3]
    offs = {}
    off = 0
    for i in so:
        offs[i] = off
        off += embs[i].shape[0]
    xcat = jnp.concatenate([embs[i] for i in so], axis=0)
    a1, a2 = _split_a(l1)
    ha, hda, hua, s1a, s2a = _proj(xcat, [l1["W"], l1["Wd"], l1["Wu"]], a1, a2)

    kinds = {0: "h d", 1: "h d u", 2: "h u", 3: "u"}
    h2, hd2, hu2, s12, s22 = {}, {}, {}, {}, {}
    for i in (0, 1, 2, 3):
        toks = kinds[i].split()
        ws = [l2[wmap[t]] for t in toks]
        sc = "h" in toks
        outs = _attn(laps[i], ha, s1a, s2a, offs[i],
                     bnds[i] if i > 0 else None, hda,
                     offs[i - 1] if i > 0 else 0,
                     bnds[i + 1] if i < 3 else None, hua,
                     offs[i + 1] if i < 3 else 0,
                     ws, a2p[0] if sc else None, a2p[1] if sc else None)
        dsts = {"h": h2, "d": hd2, "u": hu2}
        for t, o in zip(toks, outs[:len(toks)]):
            dsts[t][i] = o
        if sc:
            s12[i], s22[i] = outs[-2], outs[-1]

    kinds = {0: "h d", 1: "h u", 2: "u"}
    h3, hd3, hu3, s13, s23 = {}, {}, {}, {}, {}
    for i in (0, 1, 2):
        toks = kinds[i].split()
        ws = [l3[wmap[t]] for t in toks]
        sc = "h" in toks
        outs = _attn(laps[i], h2[i], s12[i], s22[i], 0,
                     bnds[i] if i > 0 else None, hd2.get(i - 1), 0,
                     bnds[i + 1], hu2.get(i + 1), 0,
                     ws, a3p[0] if sc else None, a3p[1] if sc else None)
        dsts = {"h": h3, "d": hd3, "u": hu3}
        for t, o in zip(toks, outs[:len(toks)]):
            dsts[t][i] = o
        if sc:
            s13[i], s23[i] = outs[-2], outs[-1]

    h40, s14, s24 = _attn(laps[0], h3[0], s13[0], s23[0], 0,
                          None, None, 0, bnds[1], hu3[1], 0,
                          [l4["W"]], a4p[0], a4p[1])
    (bgpu,) = _attn(laps[1], h3[1], s13[1], s23[1], 0,
                    bnds[1], hd3[0], 0, bnds[2], hu3[2], 0,
                    [l4["Wu"]], bg=Bg, emit_out=False)

    rows = _final(Lg, idx.reshape(-1, 1), s14, s24, h40, bgpu,
                  params["W_rel"], params["b_rel"].reshape(1, -1))

    nz = jnp.stack(jnp.nonzero(rel, size=rel.shape[0]), axis=1)
    return rows[nz]
